# baseline (speedup 1.0000x reference)
import functools

import jax
import jax.numpy as jnp
from jax import lax
from jax.experimental import pallas as pl
from jax.experimental.pallas import tpu as pltpu
from jax.experimental.pallas import tpu_sc as plsc

NC = 2
NS = 16
NWK = NC * NS


def _silu(v):
    return v * jax.nn.sigmoid(v)


def _pick_block(n, cands):
    for c in cands:
        if n % c == 0:
            return c
    return n


def _prep_body(x_ref, w1a_ref, w1b_ref, ta_ref, tb_ref):
    xb = x_ref[...]
    ta_ref[...] = jnp.dot(xb, w1a_ref[...], preferred_element_type=jnp.float32)
    tb_ref[...] = jnp.dot(xb, w1b_ref[...], preferred_element_type=jnp.float32)


def _prep_call(x, w1a, w1b, bn):
    n, d = x.shape
    return pl.pallas_call(
        _prep_body,
        grid=(n // bn,),
        in_specs=[
            pl.BlockSpec((bn, d), lambda i: (i, 0)),
            pl.BlockSpec((d, d), lambda i: (0, 0)),
            pl.BlockSpec((d, d), lambda i: (0, 0)),
        ],
        out_specs=[
            pl.BlockSpec((bn, d), lambda i: (i, 0)),
            pl.BlockSpec((bn, d), lambda i: (i, 0)),
        ],
        out_shape=[
            jax.ShapeDtypeStruct((n, d), jnp.float32),
            jax.ShapeDtypeStruct((n, d), jnp.float32),
        ],
    )(x, w1a, w1b)


def _make_gather(epad, ew, d, np4):
    mesh = plsc.VectorSubcoreMesh(
        core_axis_name="c", subcore_axis_name="s", num_cores=NC, num_subcores=NS)

    @functools.partial(
        pl.kernel,
        out_type=(jax.ShapeDtypeStruct((epad, d), jnp.float32),
                  jax.ShapeDtypeStruct((4, epad), jnp.float32)),
        mesh=mesh,
        scratch_types=[
            pltpu.VMEM((8, 128), jnp.int32),
            pltpu.VMEM((8, 128), jnp.int32),
            pltpu.VMEM((2, 128, 128), jnp.float32),
            pltpu.VMEM((np4,), jnp.float32),
            pltpu.VMEM((4, 1024), jnp.float32),
            pltpu.SemaphoreType.DMA,
            pltpu.SemaphoreType.DMA,
            pltpu.SemaphoreType.DMA,
        ],
        compiler_params=pltpu.CompilerParams(needs_layout_passes=False),
    )
    def gk(ta, tb, dsti, srci, posf, gab, geo,
           idxd, idxs, bufa, posv, gbuf, sema, semb, semw):
        c = lax.axis_index("c")
        s = lax.axis_index("s")
        base = (s * NC + c) * ew
        pltpu.sync_copy(posf, posv)

        def body(i, carry):
            off = pl.multiple_of(base + i * 1024, 1024)
            r0 = pl.multiple_of(off // 128, 8)
            pltpu.sync_copy(dsti.at[pl.ds(r0, 8)], idxd)
            pltpu.sync_copy(srci.at[pl.ds(r0, 8)], idxs)
            def issue_a(q):
                return pltpu.async_copy(
                    ta.at[idxd.at[q]], bufa.at[q % 2], sema)

            def issue_b(q):
                return pltpu.async_copy(
                    tb.at[idxs.at[q]], bufa.at[q % 2], semb, add=True)

            def issue_w(q):
                return pltpu.async_copy(
                    bufa.at[q % 2], gab.at[pl.ds(off + q * 128, 128)], semw)

            gta = [None] * 8
            gtb = [None] * 8
            wbk = [None] * 8
            gta[0] = issue_a(0)
            for j in range(8):
                for kk in range(8):
                    lq = kk * 16
                    p = j * 128 + lq
                    di = idxd[j, pl.ds(lq, 16)] * 4
                    si = idxs[j, pl.ds(lq, 16)] * 4
                    dx = (plsc.load_gather(posv, [di])
                          - plsc.load_gather(posv, [si]))
                    dy = (plsc.load_gather(posv, [di + 1])
                          - plsc.load_gather(posv, [si + 1]))
                    dz = (plsc.load_gather(posv, [di + 2])
                          - plsc.load_gather(posv, [si + 2]))
                    gbuf[0, pl.ds(p, 16)] = dx
                    gbuf[1, pl.ds(p, 16)] = dy
                    gbuf[2, pl.ds(p, 16)] = dz
                    gbuf[3, pl.ds(p, 16)] = dx * dx + dy * dy + dz * dz
            pltpu.sync_copy(gbuf, geo.at[:, pl.ds(off, 1024)])
            for q in range(8):
                gta[q].wait()
                gtb[q] = issue_b(q)
                if q + 1 < 8:
                    if q >= 1:
                        wbk[q - 1].wait()
                    gta[q + 1] = issue_a(q + 1)
                gtb[q].wait()
                wbk[q] = issue_w(q)
            wbk[6].wait()
            wbk[7].wait()
            return carry

        lax.fori_loop(0, ew // 1024, body, 0)

    return gk


def _edge_body(gab_ref, geo_ref, ea_ref, w1c_ref, w1d_ref, b1_ref,
               w2_ref, b2_ref, wc_ref, bc_ref, m_ref, gd_ref):
    geo = geo_ref[...]
    i4a = lax.broadcasted_iota(jnp.int32, (4, 4), 0)
    i4b = lax.broadcasted_iota(jnp.int32, (4, 4), 1)
    eye4 = (i4a == i4b).astype(jnp.float32)
    d4 = lax.dot_general(geo, eye4, (((0,), (0,)), ((), ())),
                         preferred_element_type=jnp.float32)
    r2 = d4[:, 3:4]
    rinv = lax.rsqrt(r2 + 1e-8)
    pre = (gab_ref[...] + r2 * w1c_ref[...] + b1_ref[...]
           + jnp.dot(ea_ref[...], w1d_ref[...],
                     preferred_element_type=jnp.float32))
    h = _silu(pre)
    m = _silu(jnp.dot(h, w2_ref[...], preferred_element_type=jnp.float32)
              + b2_ref[...])
    m_ref[...] = m
    gamma = jnp.sum(m * wc_ref[...], axis=1, keepdims=True) + bc_ref[...]
    lane4 = lax.broadcasted_iota(jnp.int32, d4.shape, 1)
    gd4 = jnp.where(lane4 == 3, 1.0, gamma * rinv * d4)
    be = d4.shape[0]
    gd_ref[...] = jnp.concatenate(
        [gd4, jnp.zeros((be, 124), jnp.float32)], axis=1)


def _edge_call(gab, geo, ea, w1c, w1d, b1r, w2, b2r, wcr, bcr, be):
    epad, d = gab.shape
    e, ed = ea.shape
    h = w2.shape[0]
    elast = e // be - 1
    return pl.pallas_call(
        _edge_body,
        grid=(epad // be,),
        in_specs=[
            pl.BlockSpec((be, d), lambda i: (i, 0)),
            pl.BlockSpec((4, be), lambda i: (0, i)),
            pl.BlockSpec((be, ed), lambda i: (jnp.minimum(i, elast), 0)),
            pl.BlockSpec((1, h), lambda i: (0, 0)),
            pl.BlockSpec((ed, h), lambda i: (0, 0)),
            pl.BlockSpec((1, h), lambda i: (0, 0)),
            pl.BlockSpec((h, h), lambda i: (0, 0)),
            pl.BlockSpec((1, h), lambda i: (0, 0)),
            pl.BlockSpec((1, h), lambda i: (0, 0)),
            pl.BlockSpec((1, 1), lambda i: (0, 0)),
        ],
        out_specs=[
            pl.BlockSpec((be, d), lambda i: (i, 0)),
            pl.BlockSpec((be, d), lambda i: (i, 0)),
        ],
        out_shape=[
            jax.ShapeDtypeStruct((epad, d), jnp.float32),
            jax.ShapeDtypeStruct((epad, d), jnp.float32),
        ],
    )(gab, geo, ea, w1c, w1d, b1r, w2, b2r, wcr, bcr)


def _make_scatter(epad, npad, rw, d):
    ew2 = epad // NS
    mesh = plsc.VectorSubcoreMesh(
        core_axis_name="c", subcore_axis_name="s", num_cores=NC, num_subcores=NS)

    @functools.partial(
        pl.kernel,
        out_type=jax.ShapeDtypeStruct((NC * npad, d), jnp.float32),
        mesh=mesh,
        scratch_types=[
            pltpu.VMEM((8, 128), jnp.int32),
            pltpu.VMEM((2, 128, d), jnp.float32),
            pltpu.VMEM_SHARED((npad, d), jnp.float32),
            pltpu.SemaphoreType.DMA,
            pltpu.SemaphoreType.DMA,
        ],
    )
    def sk(mv, gv, dsti, zer, out, idx, buf, acc, seml, sems):
        c = lax.axis_index("c")
        s = lax.axis_index("s")
        base = s * ew2
        srw = pl.multiple_of(s * rw, 8)
        pltpu.sync_copy(zer.at[pl.ds(srw, rw)], acc.at[pl.ds(srw, rw)])
        plsc.subcore_barrier()

        def mk_body(data):
            def body(i, carry):
                off = pl.multiple_of(base + i * 1024, 1024)
                r0 = pl.multiple_of(off // 128, 8)
                pltpu.sync_copy(dsti.at[pl.ds(r0, 8)], idx)

                def issue_l(q):
                    return pltpu.async_copy(
                        data.at[pl.ds(off + q * 128, 128)],
                        buf.at[q % 2], seml)

                lds = [None] * 8
                sca = [None] * 8
                lds[0] = issue_l(0)
                for q in range(8):
                    if q + 1 < 8:
                        if q >= 1:
                            sca[q - 1].wait()
                        lds[q + 1] = issue_l(q + 1)
                    lds[q].wait()
                    sca[q] = pltpu.async_copy(
                        buf.at[q % 2], acc.at[idx.at[q]], sems, add=True)
                sca[6].wait()
                sca[7].wait()
                return carry
            return body

        @pl.when(c == 0)
        def _():
            lax.fori_loop(0, ew2 // 1024, mk_body(mv), 0)

        @pl.when(c == 1)
        def _():
            lax.fori_loop(0, ew2 // 1024, mk_body(gv), 0)

        plsc.subcore_barrier()
        pltpu.sync_copy(acc.at[pl.ds(srw, rw)],
                        out.at[pl.ds(pl.multiple_of(c * npad + srw, 8), rw)])

    return sk


def _node_body(x_ref, pp_ref, pm_ref, pg_ref, wn1a_ref, wn1b_ref, bn1_ref,
               wn2_ref, bn2_ref, xo_ref, po_ref):
    pg = pg_ref[...]
    lane = lax.broadcasted_iota(jnp.int32, pg.shape, 1)
    deg = jnp.sum(jnp.where(lane == 3, pg, 0.0), axis=1, keepdims=True)
    deg = jnp.maximum(deg, 1.0)
    msum = pm_ref[...] / deg
    hn = _silu(jnp.dot(x_ref[...], wn1a_ref[...],
                       preferred_element_type=jnp.float32)
               + jnp.dot(msum, wn1b_ref[...],
                         preferred_element_type=jnp.float32)
               + bn1_ref[...])
    xo_ref[...] = (jnp.dot(hn, wn2_ref[...], preferred_element_type=jnp.float32)
                   + bn2_ref[...])
    pg16 = pg[:, :16]
    lane16 = lax.broadcasted_iota(jnp.int32, pg16.shape, 1)
    po_ref[...] = pp_ref[...] + jnp.where(lane16 < 3, pg16 / deg, 0.0)


def _node_call(x, pp, pm, pg, wn1a, wn1b, bn1r, wn2, bn2r, bn):
    n, d = x.shape
    h = wn2.shape[0]
    return pl.pallas_call(
        _node_body,
        grid=(n // bn,),
        in_specs=[
            pl.BlockSpec((bn, d), lambda i: (i, 0)),
            pl.BlockSpec((bn, 16), lambda i: (i, 0)),
            pl.BlockSpec((bn, d), lambda i: (i, 0)),
            pl.BlockSpec((bn, d), lambda i: (i, 0)),
            pl.BlockSpec((d, h), lambda i: (0, 0)),
            pl.BlockSpec((h, h), lambda i: (0, 0)),
            pl.BlockSpec((1, h), lambda i: (0, 0)),
            pl.BlockSpec((h, d), lambda i: (0, 0)),
            pl.BlockSpec((1, d), lambda i: (0, 0)),
        ],
        out_specs=[
            pl.BlockSpec((bn, d), lambda i: (i, 0)),
            pl.BlockSpec((bn, 16), lambda i: (i, 0)),
        ],
        out_shape=[
            jax.ShapeDtypeStruct((n, d), jnp.float32),
            jax.ShapeDtypeStruct((n, 16), jnp.float32),
        ],
    )(x, pp, pm, pg, wn1a, wn1b, bn1r, wn2, bn2r)


def kernel(x, pos, edge_index, edge_attr, W1, b1, W2, b2, Wn1, bn1, Wn2, bn2,
           Wc, bc):
    n, d = x.shape
    e = edge_index.shape[1]
    h = W2.shape[0]

    ew = -(-e // (NWK * 1024)) * 1024
    epad = ew * NWK
    npad = -(-n // 128) * 128
    if npad == n:
        npad += 128
    rw = npad // NS
    np4 = -(-(4 * n) // 128) * 128

    pp = jnp.pad(pos, ((0, 0), (0, 16 - pos.shape[1])))
    posf = jnp.pad(pos, ((0, 0), (0, 1))).reshape(-1)
    posf = jnp.pad(posf, (0, np4 - posf.shape[0]))
    src = edge_index[0]
    dst = edge_index[1]
    pe = epad - e
    dst_g = jnp.concatenate([dst, jnp.zeros((pe,), jnp.int32)]).reshape(-1, 128)
    src_g = jnp.concatenate([src, jnp.zeros((pe,), jnp.int32)]).reshape(-1, 128)
    dst_s = jnp.concatenate(
        [dst, jnp.full((pe,), npad - 1, jnp.int32)]).reshape(-1, 128)
    w1a = W1[:d]
    w1b = W1[d:2 * d]
    w1c = W1[2 * d:2 * d + 1]
    w1d = W1[2 * d + 1:]
    b1r = b1.reshape(1, h)
    b2r = b2.reshape(1, h)
    wcr = Wc.reshape(1, h)
    bcr = bc.reshape(1, 1)
    wn1a = Wn1[:d]
    wn1b = Wn1[d:]
    bn1r = bn1.reshape(1, h)
    bn2r = bn2.reshape(1, d)
    zer = jnp.zeros((npad, d), jnp.float32)

    bn = _pick_block(n, (1024, 1000, 512, 500, 256, 250, 200, 128, 8))
    be = 512 if (e % 512 == 0 and epad % 512 == 0) else 128
    assert e % be == 0 and epad % be == 0

    ta, tb = _prep_call(x, w1a, w1b, bn)
    gab, geo = _make_gather(epad, ew, d, np4)(ta, tb, dst_g, src_g, posf)
    m, gd = _edge_call(gab, geo, edge_attr, w1c, w1d, b1r, W2, b2r, wcr,
                       bcr, be)
    parts = _make_scatter(epad, npad, rw, d)(m, gd, dst_s, zer)
    pm = parts[:n]
    pg = parts[npad:npad + n]
    xo, po = _node_call(x, pp, pm, pg, wn1a, wn1b, bn1r, Wn2, bn2r, bn)
    return (xo, po[:, :3])


# speedup vs baseline: 1.1429x; 1.1429x over previous
import functools

import jax
import jax.numpy as jnp
from jax import lax
from jax.experimental import pallas as pl
from jax.experimental.pallas import tpu as pltpu
from jax.experimental.pallas import tpu_sc as plsc

NC = 2
NS = 16
NWK = NC * NS


def _silu(v):
    return v * jax.nn.sigmoid(v)


def _pick_block(n, cands):
    for c in cands:
        if n % c == 0:
            return c
    return n


def _prep_body(x_ref, w1a_ref, w1b_ref, ta_ref, tb_ref):
    xb = x_ref[...]
    ta_ref[...] = jnp.dot(xb, w1a_ref[...], preferred_element_type=jnp.float32)
    tb_ref[...] = jnp.dot(xb, w1b_ref[...], preferred_element_type=jnp.float32)


def _prep_call(x, w1a, w1b, bn):
    n, d = x.shape
    return pl.pallas_call(
        _prep_body,
        grid=(n // bn,),
        in_specs=[
            pl.BlockSpec((bn, d), lambda i: (i, 0)),
            pl.BlockSpec((d, d), lambda i: (0, 0)),
            pl.BlockSpec((d, d), lambda i: (0, 0)),
        ],
        out_specs=[
            pl.BlockSpec((bn, d), lambda i: (i, 0)),
            pl.BlockSpec((bn, d), lambda i: (i, 0)),
        ],
        out_shape=[
            jax.ShapeDtypeStruct((n, d), jnp.float32),
            jax.ShapeDtypeStruct((n, d), jnp.float32),
        ],
    )(x, w1a, w1b)


def _make_gather(epad, d, np4, ch0):
    chp = epad // NS // 1024
    ch1 = chp - ch0
    mesh = plsc.VectorSubcoreMesh(
        core_axis_name="c", subcore_axis_name="s", num_cores=NC, num_subcores=NS)

    @functools.partial(
        pl.kernel,
        out_type=(jax.ShapeDtypeStruct((epad, d), jnp.float32),
                  jax.ShapeDtypeStruct((epad, d), jnp.float32),
                  jax.ShapeDtypeStruct((4, epad), jnp.float32)),
        mesh=mesh,
        scratch_types=[
            pltpu.VMEM((8, 128), jnp.int32),
            pltpu.VMEM((8, 128), jnp.int32),
            pltpu.VMEM((2, 128, 128), jnp.float32),
            pltpu.VMEM((2, 128, 128), jnp.float32),
            pltpu.VMEM((np4,), jnp.float32),
            pltpu.VMEM((4, 1024), jnp.float32),
            pltpu.SemaphoreType.DMA,
            pltpu.SemaphoreType.DMA,
            pltpu.SemaphoreType.DMA,
        ],
        compiler_params=pltpu.CompilerParams(needs_layout_passes=False),
    )
    def gk(ta, tb, dsti, srci, posf, ga, gb, geo,
           idxd, idxs, bufa, bufb, posv, gbuf, sema, semb, semw):
        c = lax.axis_index("c")
        s = lax.axis_index("s")
        base = s * (chp * 1024) + c * (ch0 * 1024)
        nch = jnp.where(c == 0, ch0, ch1)
        pltpu.sync_copy(posf, posv)

        def body(i, carry):
            off = pl.multiple_of(base + i * 1024, 1024)
            r0 = pl.multiple_of(off // 128, 8)
            pltpu.sync_copy(dsti.at[pl.ds(r0, 8)], idxd)
            pltpu.sync_copy(srci.at[pl.ds(r0, 8)], idxs)
            def issue_g(q):
                b = q % 2
                return (
                    pltpu.async_copy(ta.at[idxd.at[q]], bufa.at[b], sema),
                    pltpu.async_copy(tb.at[idxs.at[q]], bufb.at[b], semb),
                )

            def issue_w(q):
                b = q % 2
                return (
                    pltpu.async_copy(
                        bufa.at[b], ga.at[pl.ds(off + q * 128, 128)], semw),
                    pltpu.async_copy(
                        bufb.at[b], gb.at[pl.ds(off + q * 128, 128)], semw),
                )

            gth = [None] * 8
            wbk = [None] * 8
            gth[0] = issue_g(0)
            for j in range(8):
                for kk in range(8):
                    lq = kk * 16
                    p = j * 128 + lq
                    di = idxd[j, pl.ds(lq, 16)] * 4
                    si = idxs[j, pl.ds(lq, 16)] * 4
                    dx = (plsc.load_gather(posv, [di])
                          - plsc.load_gather(posv, [si]))
                    dy = (plsc.load_gather(posv, [di + 1])
                          - plsc.load_gather(posv, [si + 1]))
                    dz = (plsc.load_gather(posv, [di + 2])
                          - plsc.load_gather(posv, [si + 2]))
                    gbuf[0, pl.ds(p, 16)] = dx
                    gbuf[1, pl.ds(p, 16)] = dy
                    gbuf[2, pl.ds(p, 16)] = dz
                    gbuf[3, pl.ds(p, 16)] = dx * dx + dy * dy + dz * dz
            pltpu.sync_copy(gbuf, geo.at[:, pl.ds(off, 1024)])
            for q in range(8):
                if q + 1 < 8:
                    if q >= 1:
                        wbk[q - 1][0].wait()
                        wbk[q - 1][1].wait()
                    gth[q + 1] = issue_g(q + 1)
                gth[q][0].wait()
                gth[q][1].wait()
                wbk[q] = issue_w(q)
            wbk[6][0].wait()
            wbk[6][1].wait()
            wbk[7][0].wait()
            wbk[7][1].wait()
            return carry

        lax.fori_loop(0, nch, body, 0)

    return gk


def _edge_body(ga_ref, gb_ref, geo_ref, ea_ref, w1c_ref, w1d_ref, b1_ref,
               w2_ref, b2_ref, wc_ref, bc_ref, m_ref, gd_ref):
    geo = geo_ref[...]
    i4a = lax.broadcasted_iota(jnp.int32, (4, 4), 0)
    i4b = lax.broadcasted_iota(jnp.int32, (4, 4), 1)
    eye4 = (i4a == i4b).astype(jnp.float32)
    d4 = lax.dot_general(geo, eye4, (((0,), (0,)), ((), ())),
                         preferred_element_type=jnp.float32)
    r2 = d4[:, 3:4]
    rinv = lax.rsqrt(r2 + 1e-8)
    pre = (ga_ref[...] + gb_ref[...] + r2 * w1c_ref[...] + b1_ref[...]
           + jnp.dot(ea_ref[...], w1d_ref[...],
                     preferred_element_type=jnp.float32))
    h = _silu(pre)
    m = _silu(jnp.dot(h, w2_ref[...], preferred_element_type=jnp.float32)
              + b2_ref[...])
    m_ref[...] = m
    gamma = jnp.sum(m * wc_ref[...], axis=1, keepdims=True) + bc_ref[...]
    lane4 = lax.broadcasted_iota(jnp.int32, d4.shape, 1)
    gd4 = jnp.where(lane4 == 3, 1.0, gamma * rinv * d4)
    be = d4.shape[0]
    gd_ref[...] = jnp.concatenate(
        [gd4, jnp.zeros((be, 124), jnp.float32)], axis=1)


def _edge_call(ga, gb, geo, eap, w1c, w1d, b1r, w2, b2r, wcr, bcr, be):
    epad, d = ga.shape
    ed = eap.shape[1]
    h = w2.shape[0]
    return pl.pallas_call(
        _edge_body,
        grid=(epad // be,),
        in_specs=[
            pl.BlockSpec((be, d), lambda i: (i, 0)),
            pl.BlockSpec((be, d), lambda i: (i, 0)),
            pl.BlockSpec((4, be), lambda i: (0, i)),
            pl.BlockSpec((be, ed), lambda i: (i, 0)),
            pl.BlockSpec((1, h), lambda i: (0, 0)),
            pl.BlockSpec((ed, h), lambda i: (0, 0)),
            pl.BlockSpec((1, h), lambda i: (0, 0)),
            pl.BlockSpec((h, h), lambda i: (0, 0)),
            pl.BlockSpec((1, h), lambda i: (0, 0)),
            pl.BlockSpec((1, h), lambda i: (0, 0)),
            pl.BlockSpec((1, 1), lambda i: (0, 0)),
        ],
        out_specs=[
            pl.BlockSpec((be, d), lambda i: (i, 0)),
            pl.BlockSpec((be, d), lambda i: (i, 0)),
        ],
        out_shape=[
            jax.ShapeDtypeStruct((epad, d), jnp.float32),
            jax.ShapeDtypeStruct((epad, d), jnp.float32),
        ],
    )(ga, gb, geo, eap, w1c, w1d, b1r, w2, b2r, wcr, bcr)


def _make_scatter(epad, npad, rw, d):
    ew2 = epad // NS
    mesh = plsc.VectorSubcoreMesh(
        core_axis_name="c", subcore_axis_name="s", num_cores=NC, num_subcores=NS)

    @functools.partial(
        pl.kernel,
        out_type=jax.ShapeDtypeStruct((NC * npad, d), jnp.float32),
        mesh=mesh,
        scratch_types=[
            pltpu.VMEM((8, 128), jnp.int32),
            pltpu.VMEM((2, 128, d), jnp.float32),
            pltpu.VMEM_SHARED((npad, d), jnp.float32),
            pltpu.SemaphoreType.DMA,
            pltpu.SemaphoreType.DMA,
        ],
    )
    def sk(mv, gv, dsti, zer, out, idx, buf, acc, seml, sems):
        c = lax.axis_index("c")
        s = lax.axis_index("s")
        base = s * ew2
        srw = pl.multiple_of(s * rw, 8)
        pltpu.sync_copy(zer.at[pl.ds(srw, rw)], acc.at[pl.ds(srw, rw)])
        plsc.subcore_barrier()

        def mk_body(data):
            def body(i, carry):
                off = pl.multiple_of(base + i * 1024, 1024)
                r0 = pl.multiple_of(off // 128, 8)
                pltpu.sync_copy(dsti.at[pl.ds(r0, 8)], idx)

                def issue_l(q):
                    return pltpu.async_copy(
                        data.at[pl.ds(off + q * 128, 128)],
                        buf.at[q % 2], seml)

                lds = [None] * 8
                sca = [None] * 8
                lds[0] = issue_l(0)
                for q in range(8):
                    if q + 1 < 8:
                        if q >= 1:
                            sca[q - 1].wait()
                        lds[q + 1] = issue_l(q + 1)
                    lds[q].wait()
                    sca[q] = pltpu.async_copy(
                        buf.at[q % 2], acc.at[idx.at[q]], sems, add=True)
                sca[6].wait()
                sca[7].wait()
                return carry
            return body

        @pl.when(c == 0)
        def _():
            lax.fori_loop(0, ew2 // 1024, mk_body(mv), 0)

        @pl.when(c == 1)
        def _():
            lax.fori_loop(0, ew2 // 1024, mk_body(gv), 0)

        plsc.subcore_barrier()
        pltpu.sync_copy(acc.at[pl.ds(srw, rw)],
                        out.at[pl.ds(pl.multiple_of(c * npad + srw, 8), rw)])

    return sk


def _node_body(x_ref, pp_ref, pm_ref, pg_ref, wn1a_ref, wn1b_ref, bn1_ref,
               wn2_ref, bn2_ref, xo_ref, po_ref):
    pg = pg_ref[...]
    lane = lax.broadcasted_iota(jnp.int32, pg.shape, 1)
    deg = jnp.sum(jnp.where(lane == 3, pg, 0.0), axis=1, keepdims=True)
    deg = jnp.maximum(deg, 1.0)
    msum = pm_ref[...] / deg
    hn = _silu(jnp.dot(x_ref[...], wn1a_ref[...],
                       preferred_element_type=jnp.float32)
               + jnp.dot(msum, wn1b_ref[...],
                         preferred_element_type=jnp.float32)
               + bn1_ref[...])
    xo_ref[...] = (jnp.dot(hn, wn2_ref[...], preferred_element_type=jnp.float32)
                   + bn2_ref[...])
    pg16 = pg[:, :16]
    lane16 = lax.broadcasted_iota(jnp.int32, pg16.shape, 1)
    po_ref[...] = pp_ref[...] + jnp.where(lane16 < 3, pg16 / deg, 0.0)


def _node_call(x, pp, pm, pg, wn1a, wn1b, bn1r, wn2, bn2r, bn):
    n, d = x.shape
    h = wn2.shape[0]
    return pl.pallas_call(
        _node_body,
        grid=(n // bn,),
        in_specs=[
            pl.BlockSpec((bn, d), lambda i: (i, 0)),
            pl.BlockSpec((bn, 16), lambda i: (i, 0)),
            pl.BlockSpec((bn, d), lambda i: (i, 0)),
            pl.BlockSpec((bn, d), lambda i: (i, 0)),
            pl.BlockSpec((d, h), lambda i: (0, 0)),
            pl.BlockSpec((h, h), lambda i: (0, 0)),
            pl.BlockSpec((1, h), lambda i: (0, 0)),
            pl.BlockSpec((h, d), lambda i: (0, 0)),
            pl.BlockSpec((1, d), lambda i: (0, 0)),
        ],
        out_specs=[
            pl.BlockSpec((bn, d), lambda i: (i, 0)),
            pl.BlockSpec((bn, 16), lambda i: (i, 0)),
        ],
        out_shape=[
            jax.ShapeDtypeStruct((n, d), jnp.float32),
            jax.ShapeDtypeStruct((n, 16), jnp.float32),
        ],
    )(x, pp, pm, pg, wn1a, wn1b, bn1r, wn2, bn2r)


def kernel(x, pos, edge_index, edge_attr, W1, b1, W2, b2, Wn1, bn1, Wn2, bn2,
           Wc, bc):
    n, d = x.shape
    e = edge_index.shape[1]
    h = W2.shape[0]

    ew = -(-e // (NWK * 1024)) * 1024
    epad = ew * NWK
    npad = -(-n // 128) * 128
    if npad == n:
        npad += 128
    rw = npad // NS
    np4 = -(-(4 * n) // 128) * 128

    pp = jnp.pad(pos, ((0, 0), (0, 16 - pos.shape[1])))
    posf = jnp.pad(pos, ((0, 0), (0, 1))).reshape(-1)
    posf = jnp.pad(posf, (0, np4 - posf.shape[0]))
    src = edge_index[0]
    dst = edge_index[1]
    pe = epad - e
    dst_g = jnp.concatenate([dst, jnp.zeros((pe,), jnp.int32)]).reshape(-1, 128)
    src_g = jnp.concatenate([src, jnp.zeros((pe,), jnp.int32)]).reshape(-1, 128)
    dst_s = jnp.concatenate(
        [dst, jnp.full((pe,), npad - 1, jnp.int32)]).reshape(-1, 128)
    eap = jnp.pad(edge_attr, ((0, pe), (0, 0)))
    w1a = W1[:d]
    w1b = W1[d:2 * d]
    w1c = W1[2 * d:2 * d + 1]
    w1d = W1[2 * d + 1:]
    b1r = b1.reshape(1, h)
    b2r = b2.reshape(1, h)
    wcr = Wc.reshape(1, h)
    bcr = bc.reshape(1, 1)
    wn1a = Wn1[:d]
    wn1b = Wn1[d:]
    bn1r = bn1.reshape(1, h)
    bn2r = bn2.reshape(1, d)
    zer = jnp.zeros((npad, d), jnp.float32)

    bn = _pick_block(n, (1024, 1000, 512, 500, 256, 250, 200, 128, 8))
    be = _pick_block(epad, (1024, 512, 256, 128))

    ta, tb = _prep_call(x, w1a, w1b, bn)
    chp = epad // NS // 1024
    ga, gb, geo = _make_gather(epad, d, np4, chp * 7 // 20)(
        ta, tb, dst_g, src_g, posf)
    m, gd = _edge_call(ga, gb, geo, eap, w1c, w1d, b1r, W2, b2r, wcr, bcr, be)
    parts = _make_scatter(epad, npad, rw, d)(m, gd, dst_s, zer)
    pm = parts[:n]
    pg = parts[npad:npad + n]
    xo, po = _node_call(x, pp, pm, pg, wn1a, wn1b, bn1r, Wn2, bn2r, bn)
    return (xo, po[:, :3])


# speedup vs baseline: 1.1710x; 1.0246x over previous
import functools

import jax
import jax.numpy as jnp
from jax import lax
from jax.experimental import pallas as pl
from jax.experimental.pallas import tpu as pltpu
from jax.experimental.pallas import tpu_sc as plsc

NC = 2
NS = 16
NWK = NC * NS


def _silu(v):
    return v * jax.nn.sigmoid(v)


def _pick_block(n, cands):
    for c in cands:
        if n % c == 0:
            return c
    return n


def _prep_body(x_ref, w1a_ref, w1b_ref, ta_ref, tb_ref):
    xb = x_ref[...]
    ta_ref[...] = jnp.dot(xb, w1a_ref[...], preferred_element_type=jnp.float32)
    tb_ref[...] = jnp.dot(xb, w1b_ref[...], preferred_element_type=jnp.float32)


def _prep_call(x, w1a, w1b, bn):
    n, d = x.shape
    return pl.pallas_call(
        _prep_body,
        grid=(n // bn,),
        in_specs=[
            pl.BlockSpec((bn, d), lambda i: (i, 0)),
            pl.BlockSpec((d, d), lambda i: (0, 0)),
            pl.BlockSpec((d, d), lambda i: (0, 0)),
        ],
        out_specs=[
            pl.BlockSpec((bn, d), lambda i: (i, 0)),
            pl.BlockSpec((bn, d), lambda i: (i, 0)),
        ],
        out_shape=[
            jax.ShapeDtypeStruct((n, d), jnp.float32),
            jax.ShapeDtypeStruct((n, d), jnp.float32),
        ],
    )(x, w1a, w1b)


def _make_gather(epad, d, np4, ch0):
    chp = epad // NS // 1024
    ch1 = chp - ch0
    mesh = plsc.VectorSubcoreMesh(
        core_axis_name="c", subcore_axis_name="s", num_cores=NC, num_subcores=NS)

    @functools.partial(
        pl.kernel,
        out_type=(jax.ShapeDtypeStruct((epad, d), jnp.float32),
                  jax.ShapeDtypeStruct((epad, d), jnp.float32),
                  jax.ShapeDtypeStruct((4, epad), jnp.float32)),
        mesh=mesh,
        scratch_types=[
            pltpu.VMEM((8, 128), jnp.int32),
            pltpu.VMEM((8, 128), jnp.int32),
            pltpu.VMEM((2, 128, 128), jnp.float32),
            pltpu.VMEM((2, 128, 128), jnp.float32),
            pltpu.VMEM((np4,), jnp.float32),
            pltpu.VMEM((4, 1024), jnp.float32),
            pltpu.SemaphoreType.DMA,
            pltpu.SemaphoreType.DMA,
            pltpu.SemaphoreType.DMA,
        ],
        compiler_params=pltpu.CompilerParams(needs_layout_passes=False),
    )
    def gk(ta, tb, dsti, srci, posf, ga, gb, geo,
           idxd, idxs, bufa, bufb, posv, gbuf, sema, semb, semw):
        c = lax.axis_index("c")
        s = lax.axis_index("s")
        base = s * (chp * 1024) + c * (ch0 * 1024)
        nch = jnp.where(c == 0, ch0, ch1)
        pltpu.sync_copy(posf, posv)

        def body(i, carry):
            off = pl.multiple_of(base + i * 1024, 1024)
            r0 = pl.multiple_of(off // 128, 8)
            pltpu.sync_copy(dsti.at[pl.ds(r0, 8)], idxd)
            pltpu.sync_copy(srci.at[pl.ds(r0, 8)], idxs)
            def issue_g(q):
                b = q % 2
                return (
                    pltpu.async_copy(ta.at[idxd.at[q]], bufa.at[b], sema),
                    pltpu.async_copy(tb.at[idxs.at[q]], bufb.at[b], semb),
                )

            def issue_w(q):
                b = q % 2
                return (
                    pltpu.async_copy(
                        bufa.at[b], ga.at[pl.ds(off + q * 128, 128)], semw),
                    pltpu.async_copy(
                        bufb.at[b], gb.at[pl.ds(off + q * 128, 128)], semw),
                )

            gth = [None] * 8
            wbk = [None] * 8
            gth[0] = issue_g(0)
            for j in range(8):
                for kk in range(8):
                    lq = kk * 16
                    p = j * 128 + lq
                    di = idxd[j, pl.ds(lq, 16)] * 4
                    si = idxs[j, pl.ds(lq, 16)] * 4
                    dx = (plsc.load_gather(posv, [di])
                          - plsc.load_gather(posv, [si]))
                    dy = (plsc.load_gather(posv, [di + 1])
                          - plsc.load_gather(posv, [si + 1]))
                    dz = (plsc.load_gather(posv, [di + 2])
                          - plsc.load_gather(posv, [si + 2]))
                    gbuf[0, pl.ds(p, 16)] = dx
                    gbuf[1, pl.ds(p, 16)] = dy
                    gbuf[2, pl.ds(p, 16)] = dz
                    gbuf[3, pl.ds(p, 16)] = dx * dx + dy * dy + dz * dz
            pltpu.sync_copy(gbuf, geo.at[:, pl.ds(off, 1024)])
            for q in range(8):
                if q + 1 < 8:
                    if q >= 1:
                        wbk[q - 1][0].wait()
                        wbk[q - 1][1].wait()
                    gth[q + 1] = issue_g(q + 1)
                gth[q][0].wait()
                gth[q][1].wait()
                wbk[q] = issue_w(q)
            wbk[6][0].wait()
            wbk[6][1].wait()
            wbk[7][0].wait()
            wbk[7][1].wait()
            return carry

        lax.fori_loop(0, nch, body, 0)

    return gk


def _edge_body(ga_ref, gb_ref, geo_ref, ea_ref, w1c_ref, w1d_ref, b1_ref,
               w2_ref, b2_ref, wc_ref, bc_ref, m_ref, gd_ref):
    geo = geo_ref[...]
    i4a = lax.broadcasted_iota(jnp.int32, (4, 4), 0)
    i4b = lax.broadcasted_iota(jnp.int32, (4, 4), 1)
    eye4 = (i4a == i4b).astype(jnp.float32)
    d4 = lax.dot_general(geo, eye4, (((0,), (0,)), ((), ())),
                         preferred_element_type=jnp.float32)
    r2 = d4[:, 3:4]
    rinv = lax.rsqrt(r2 + 1e-8)
    pre = (ga_ref[...] + gb_ref[...] + r2 * w1c_ref[...] + b1_ref[...]
           + jnp.dot(ea_ref[...], w1d_ref[...],
                     preferred_element_type=jnp.float32))
    h = _silu(pre)
    m = _silu(jnp.dot(h, w2_ref[...], preferred_element_type=jnp.float32)
              + b2_ref[...])
    m_ref[...] = m
    gamma = jnp.sum(m * wc_ref[...], axis=1, keepdims=True) + bc_ref[...]
    lane4 = lax.broadcasted_iota(jnp.int32, d4.shape, 1)
    gd4 = jnp.where(lane4 == 3, 1.0, gamma * rinv * d4)
    be = d4.shape[0]
    gd_ref[...] = jnp.concatenate(
        [gd4, jnp.zeros((be, 124), jnp.float32)], axis=1)


def _edge_call(ga, gb, geo, eap, w1c, w1d, b1r, w2, b2r, wcr, bcr, be):
    epad, d = ga.shape
    ed = eap.shape[1]
    h = w2.shape[0]
    return pl.pallas_call(
        _edge_body,
        grid=(epad // be,),
        in_specs=[
            pl.BlockSpec((be, d), lambda i: (i, 0)),
            pl.BlockSpec((be, d), lambda i: (i, 0)),
            pl.BlockSpec((4, be), lambda i: (0, i)),
            pl.BlockSpec((be, ed), lambda i: (i, 0)),
            pl.BlockSpec((1, h), lambda i: (0, 0)),
            pl.BlockSpec((ed, h), lambda i: (0, 0)),
            pl.BlockSpec((1, h), lambda i: (0, 0)),
            pl.BlockSpec((h, h), lambda i: (0, 0)),
            pl.BlockSpec((1, h), lambda i: (0, 0)),
            pl.BlockSpec((1, h), lambda i: (0, 0)),
            pl.BlockSpec((1, 1), lambda i: (0, 0)),
        ],
        out_specs=[
            pl.BlockSpec((be, d), lambda i: (i, 0)),
            pl.BlockSpec((be, d), lambda i: (i, 0)),
        ],
        out_shape=[
            jax.ShapeDtypeStruct((epad, d), jnp.float32),
            jax.ShapeDtypeStruct((epad, d), jnp.float32),
        ],
    )(ga, gb, geo, eap, w1c, w1d, b1r, w2, b2r, wcr, bcr)


def _make_scatter(epad, npad, rw, d):
    ew2 = epad // NS
    mesh = plsc.VectorSubcoreMesh(
        core_axis_name="c", subcore_axis_name="s", num_cores=NC, num_subcores=NS)

    @functools.partial(
        pl.kernel,
        out_type=jax.ShapeDtypeStruct((NC * npad, d), jnp.float32),
        mesh=mesh,
        scratch_types=[
            pltpu.VMEM((8, 128), jnp.int32),
            pltpu.VMEM((2, 128, d), jnp.float32),
            pltpu.VMEM_SHARED((npad, d), jnp.float32),
            pltpu.SemaphoreType.DMA,
            pltpu.SemaphoreType.DMA,
        ],
    )
    def sk(mv, gv, dsti, zer, out, idx, buf, acc, seml, sems):
        c = lax.axis_index("c")
        s = lax.axis_index("s")
        base = s * ew2
        srw = pl.multiple_of(s * rw, 8)
        pltpu.sync_copy(zer.at[pl.ds(srw, rw)], acc.at[pl.ds(srw, rw)])
        plsc.subcore_barrier()

        def mk_body(data):
            def body(i, carry):
                off = pl.multiple_of(base + i * 1024, 1024)
                r0 = pl.multiple_of(off // 128, 8)
                pltpu.sync_copy(dsti.at[pl.ds(r0, 8)], idx)

                def issue_l(q):
                    return pltpu.async_copy(
                        data.at[pl.ds(off + q * 128, 128)],
                        buf.at[q % 2], seml)

                lds = [None] * 8
                sca = [None] * 8
                lds[0] = issue_l(0)
                for q in range(8):
                    if q + 1 < 8:
                        if q >= 1:
                            sca[q - 1].wait()
                        lds[q + 1] = issue_l(q + 1)
                    lds[q].wait()
                    sca[q] = pltpu.async_copy(
                        buf.at[q % 2], acc.at[idx.at[q]], sems, add=True)
                sca[6].wait()
                sca[7].wait()
                return carry
            return body

        @pl.when(c == 0)
        def _():
            lax.fori_loop(0, ew2 // 1024, mk_body(mv), 0)

        @pl.when(c == 1)
        def _():
            lax.fori_loop(0, ew2 // 1024, mk_body(gv), 0)

        plsc.subcore_barrier()
        pltpu.sync_copy(acc.at[pl.ds(srw, rw)],
                        out.at[pl.ds(pl.multiple_of(c * npad + srw, 8), rw)])

    return sk


def _node_body(x_ref, pp_ref, pm_ref, pg_ref, wn1a_ref, wn1b_ref, bn1_ref,
               wn2_ref, bn2_ref, xo_ref, po_ref):
    pg = pg_ref[...]
    lane = lax.broadcasted_iota(jnp.int32, pg.shape, 1)
    deg = jnp.sum(jnp.where(lane == 3, pg, 0.0), axis=1, keepdims=True)
    deg = jnp.maximum(deg, 1.0)
    msum = pm_ref[...] / deg
    hn = _silu(jnp.dot(x_ref[...], wn1a_ref[...],
                       preferred_element_type=jnp.float32)
               + jnp.dot(msum, wn1b_ref[...],
                         preferred_element_type=jnp.float32)
               + bn1_ref[...])
    xo_ref[...] = (jnp.dot(hn, wn2_ref[...], preferred_element_type=jnp.float32)
                   + bn2_ref[...])
    pg16 = pg[:, :16]
    lane16 = lax.broadcasted_iota(jnp.int32, pg16.shape, 1)
    po_ref[...] = pp_ref[...] + jnp.where(lane16 < 3, pg16 / deg, 0.0)


def _node_call(x, pp, pm, pg, wn1a, wn1b, bn1r, wn2, bn2r, bn):
    n, d = x.shape
    h = wn2.shape[0]
    return pl.pallas_call(
        _node_body,
        grid=(n // bn,),
        in_specs=[
            pl.BlockSpec((bn, d), lambda i: (i, 0)),
            pl.BlockSpec((bn, 16), lambda i: (i, 0)),
            pl.BlockSpec((bn, d), lambda i: (i, 0)),
            pl.BlockSpec((bn, d), lambda i: (i, 0)),
            pl.BlockSpec((d, h), lambda i: (0, 0)),
            pl.BlockSpec((h, h), lambda i: (0, 0)),
            pl.BlockSpec((1, h), lambda i: (0, 0)),
            pl.BlockSpec((h, d), lambda i: (0, 0)),
            pl.BlockSpec((1, d), lambda i: (0, 0)),
        ],
        out_specs=[
            pl.BlockSpec((bn, d), lambda i: (i, 0)),
            pl.BlockSpec((bn, 16), lambda i: (i, 0)),
        ],
        out_shape=[
            jax.ShapeDtypeStruct((n, d), jnp.float32),
            jax.ShapeDtypeStruct((n, 16), jnp.float32),
        ],
    )(x, pp, pm, pg, wn1a, wn1b, bn1r, wn2, bn2r)


def kernel(x, pos, edge_index, edge_attr, W1, b1, W2, b2, Wn1, bn1, Wn2, bn2,
           Wc, bc):
    n, d = x.shape
    e = edge_index.shape[1]
    h = W2.shape[0]

    ew = -(-e // (NWK * 1024)) * 1024
    epad = ew * NWK
    npad = -(-n // 128) * 128
    if npad == n:
        npad += 128
    rw = npad // NS
    np4 = -(-(4 * n) // 128) * 128

    pp = jnp.pad(pos, ((0, 0), (0, 16 - pos.shape[1])))
    posf = jnp.pad(pos, ((0, 0), (0, 1))).reshape(-1)
    posf = jnp.pad(posf, (0, np4 - posf.shape[0]))
    src = edge_index[0]
    dst = edge_index[1]
    pe = epad - e
    dst_g = jnp.concatenate([dst, jnp.zeros((pe,), jnp.int32)]).reshape(-1, 128)
    src_g = jnp.concatenate([src, jnp.zeros((pe,), jnp.int32)]).reshape(-1, 128)
    dst_s = jnp.concatenate(
        [dst, jnp.full((pe,), npad - 1, jnp.int32)]).reshape(-1, 128)
    eap = jnp.pad(edge_attr, ((0, pe), (0, 0)))
    w1a = W1[:d]
    w1b = W1[d:2 * d]
    w1c = W1[2 * d:2 * d + 1]
    w1d = W1[2 * d + 1:]
    b1r = b1.reshape(1, h)
    b2r = b2.reshape(1, h)
    wcr = Wc.reshape(1, h)
    bcr = bc.reshape(1, 1)
    wn1a = Wn1[:d]
    wn1b = Wn1[d:]
    bn1r = bn1.reshape(1, h)
    bn2r = bn2.reshape(1, d)
    zer = jnp.zeros((npad, d), jnp.float32)

    bn = _pick_block(n, (1024, 1000, 512, 500, 256, 250, 200, 128, 8))
    be = _pick_block(epad, (1024, 512, 256, 128))

    ta, tb = _prep_call(x, w1a, w1b, bn)
    chp = epad // NS // 1024
    ga, gb, geo = _make_gather(epad, d, np4, chp * 13 // 20)(
        ta, tb, dst_g, src_g, posf)
    m, gd = _edge_call(ga, gb, geo, eap, w1c, w1d, b1r, W2, b2r, wcr, bcr, be)
    parts = _make_scatter(epad, npad, rw, d)(m, gd, dst_s, zer)
    pm = parts[:n]
    pg = parts[npad:npad + n]
    xo, po = _node_call(x, pp, pm, pg, wn1a, wn1b, bn1r, Wn2, bn2r, bn)
    return (xo, po[:, :3])


# speedup vs baseline: 1.2556x; 1.0722x over previous
import functools

import jax
import jax.numpy as jnp
from jax import lax
from jax.experimental import pallas as pl
from jax.experimental.pallas import tpu as pltpu
from jax.experimental.pallas import tpu_sc as plsc

NC = 2
NS = 16
NWK = NC * NS


def _silu(v):
    return v * jax.nn.sigmoid(v)


def _pick_block(n, cands):
    for c in cands:
        if n % c == 0:
            return c
    return n


def _prep_body(x_ref, w1a_ref, w1b_ref, ta_ref, tb_ref):
    xb = x_ref[...]
    ta_ref[...] = jnp.dot(xb, w1a_ref[...], preferred_element_type=jnp.float32)
    tb_ref[...] = jnp.dot(xb, w1b_ref[...], preferred_element_type=jnp.float32)


def _prep_call(x, w1a, w1b, bn):
    n, d = x.shape
    return pl.pallas_call(
        _prep_body,
        grid=(n // bn,),
        in_specs=[
            pl.BlockSpec((bn, d), lambda i: (i, 0)),
            pl.BlockSpec((d, d), lambda i: (0, 0)),
            pl.BlockSpec((d, d), lambda i: (0, 0)),
        ],
        out_specs=[
            pl.BlockSpec((bn, d), lambda i: (i, 0)),
            pl.BlockSpec((bn, d), lambda i: (i, 0)),
        ],
        out_shape=[
            jax.ShapeDtypeStruct((n, d), jnp.float32),
            jax.ShapeDtypeStruct((n, d), jnp.float32),
        ],
    )(x, w1a, w1b)


def _make_gather(epad, d, np4, ch0):
    chp = epad // NS // 1024
    ch1 = chp - ch0
    mesh = plsc.VectorSubcoreMesh(
        core_axis_name="c", subcore_axis_name="s", num_cores=NC, num_subcores=NS)

    @functools.partial(
        pl.kernel,
        out_type=(jax.ShapeDtypeStruct((epad, d), jnp.float32),
                  jax.ShapeDtypeStruct((epad, d), jnp.float32),
                  jax.ShapeDtypeStruct((4, epad), jnp.float32)),
        mesh=mesh,
        scratch_types=[
            pltpu.VMEM((8, 128), jnp.int32),
            pltpu.VMEM((8, 128), jnp.int32),
            pltpu.VMEM((2, 128, 128), jnp.float32),
            pltpu.VMEM((2, 128, 128), jnp.float32),
            pltpu.VMEM((np4,), jnp.float32),
            pltpu.VMEM((4, 1024), jnp.float32),
            pltpu.SemaphoreType.DMA,
            pltpu.SemaphoreType.DMA,
            pltpu.SemaphoreType.DMA,
        ],
        compiler_params=pltpu.CompilerParams(needs_layout_passes=False),
    )
    def gk(ta, tb, dsti, srci, posf, ga, gb, geo,
           idxd, idxs, bufa, bufb, posv, gbuf, sema, semb, semw):
        c = lax.axis_index("c")
        s = lax.axis_index("s")
        base = s * (chp * 1024) + c * (ch0 * 1024)
        nch = jnp.where(c == 0, ch0, ch1)
        pltpu.sync_copy(posf, posv)

        def body(i, carry):
            off = pl.multiple_of(base + i * 1024, 1024)
            r0 = pl.multiple_of(off // 128, 8)
            pltpu.sync_copy(dsti.at[pl.ds(r0, 8)], idxd)
            pltpu.sync_copy(srci.at[pl.ds(r0, 8)], idxs)
            def issue_g(q):
                b = q % 2
                return (
                    pltpu.async_copy(ta.at[idxd.at[q]], bufa.at[b], sema),
                    pltpu.async_copy(tb.at[idxs.at[q]], bufb.at[b], semb),
                )

            def issue_w(q):
                b = q % 2
                return (
                    pltpu.async_copy(
                        bufa.at[b], ga.at[pl.ds(off + q * 128, 128)], semw),
                    pltpu.async_copy(
                        bufb.at[b], gb.at[pl.ds(off + q * 128, 128)], semw),
                )

            gth = [None] * 8
            wbk = [None] * 8
            gth[0] = issue_g(0)
            for j in range(8):
                for kk in range(8):
                    lq = kk * 16
                    p = j * 128 + lq
                    di = idxd[j, pl.ds(lq, 16)] * 4
                    si = idxs[j, pl.ds(lq, 16)] * 4
                    dx = (plsc.load_gather(posv, [di])
                          - plsc.load_gather(posv, [si]))
                    dy = (plsc.load_gather(posv, [di + 1])
                          - plsc.load_gather(posv, [si + 1]))
                    dz = (plsc.load_gather(posv, [di + 2])
                          - plsc.load_gather(posv, [si + 2]))
                    gbuf[0, pl.ds(p, 16)] = dx
                    gbuf[1, pl.ds(p, 16)] = dy
                    gbuf[2, pl.ds(p, 16)] = dz
                    gbuf[3, pl.ds(p, 16)] = dx * dx + dy * dy + dz * dz
            pltpu.sync_copy(gbuf, geo.at[:, pl.ds(off, 1024)])
            for q in range(8):
                if q + 1 < 8:
                    if q >= 1:
                        wbk[q - 1][0].wait()
                        wbk[q - 1][1].wait()
                    gth[q + 1] = issue_g(q + 1)
                gth[q][0].wait()
                gth[q][1].wait()
                wbk[q] = issue_w(q)
            wbk[6][0].wait()
            wbk[6][1].wait()
            wbk[7][0].wait()
            wbk[7][1].wait()
            return carry

        lax.fori_loop(0, nch, body, 0)

    return gk


def _edge_body(ga_ref, gb_ref, geo_ref, ea_ref, w1c_ref, w1d_ref, b1_ref,
               w2_ref, b2_ref, wc_ref, bc_ref, m_ref, gd_ref):
    geo = geo_ref[...]
    i4a = lax.broadcasted_iota(jnp.int32, (4, 4), 0)
    i4b = lax.broadcasted_iota(jnp.int32, (4, 4), 1)
    eye4 = (i4a == i4b).astype(jnp.float32)
    d4 = lax.dot_general(geo, eye4, (((0,), (0,)), ((), ())),
                         preferred_element_type=jnp.float32)
    r2 = d4[:, 3:4]
    rinv = lax.rsqrt(r2 + 1e-8)
    pre = (ga_ref[...] + gb_ref[...] + r2 * w1c_ref[...] + b1_ref[...]
           + jnp.dot(ea_ref[...], w1d_ref[...],
                     preferred_element_type=jnp.float32))
    h = _silu(pre)
    m = _silu(jnp.dot(h, w2_ref[...], preferred_element_type=jnp.float32)
              + b2_ref[...])
    m_ref[...] = m
    gamma = jnp.sum(m * wc_ref[...], axis=1, keepdims=True) + bc_ref[...]
    lane4 = lax.broadcasted_iota(jnp.int32, d4.shape, 1)
    gd4 = jnp.where(lane4 == 3, 1.0, gamma * rinv * d4)
    be = d4.shape[0]
    gd_ref[...] = jnp.concatenate(
        [gd4, jnp.zeros((be, 124), jnp.float32)], axis=1)


def _edge_call(ga, gb, geo, eap, w1c, w1d, b1r, w2, b2r, wcr, bcr, be):
    epad, d = ga.shape
    ed = eap.shape[1]
    h = w2.shape[0]
    return pl.pallas_call(
        _edge_body,
        grid=(epad // be,),
        in_specs=[
            pl.BlockSpec((be, d), lambda i: (i, 0)),
            pl.BlockSpec((be, d), lambda i: (i, 0)),
            pl.BlockSpec((4, be), lambda i: (0, i)),
            pl.BlockSpec((be, ed), lambda i: (i, 0)),
            pl.BlockSpec((1, h), lambda i: (0, 0)),
            pl.BlockSpec((ed, h), lambda i: (0, 0)),
            pl.BlockSpec((1, h), lambda i: (0, 0)),
            pl.BlockSpec((h, h), lambda i: (0, 0)),
            pl.BlockSpec((1, h), lambda i: (0, 0)),
            pl.BlockSpec((1, h), lambda i: (0, 0)),
            pl.BlockSpec((1, 1), lambda i: (0, 0)),
        ],
        out_specs=[
            pl.BlockSpec((be, d), lambda i: (i, 0)),
            pl.BlockSpec((be, d), lambda i: (i, 0)),
        ],
        out_shape=[
            jax.ShapeDtypeStruct((epad, d), jnp.float32),
            jax.ShapeDtypeStruct((epad, d), jnp.float32),
        ],
    )(ga, gb, geo, eap, w1c, w1d, b1r, w2, b2r, wcr, bcr)


def _make_scatter(epad, npad, rw, d):
    ew2 = epad // NS
    mesh = plsc.VectorSubcoreMesh(
        core_axis_name="c", subcore_axis_name="s", num_cores=NC, num_subcores=NS)

    @functools.partial(
        pl.kernel,
        out_type=jax.ShapeDtypeStruct((NC * npad, d), jnp.float32),
        mesh=mesh,
        scratch_types=[
            pltpu.VMEM((8, 128), jnp.int32),
            pltpu.VMEM((2, 128, d), jnp.float32),
            pltpu.VMEM_SHARED((npad, d), jnp.float32),
            pltpu.SemaphoreType.DMA,
            pltpu.SemaphoreType.DMA,
        ],
    )
    def sk(mv, gv, dsti, zer, out, idx, buf, acc, seml, sems):
        c = lax.axis_index("c")
        s = lax.axis_index("s")
        base = s * ew2
        srw = pl.multiple_of(s * rw, 8)
        pltpu.sync_copy(zer.at[pl.ds(srw, rw)], acc.at[pl.ds(srw, rw)])
        plsc.subcore_barrier()

        def mk_body(data):
            def body(i, carry):
                off = pl.multiple_of(base + i * 1024, 1024)
                r0 = pl.multiple_of(off // 128, 8)
                pltpu.sync_copy(dsti.at[pl.ds(r0, 8)], idx)

                def issue_l(q):
                    return pltpu.async_copy(
                        data.at[pl.ds(off + q * 128, 128)],
                        buf.at[q % 2], seml)

                lds = [None] * 8
                sca = [None] * 8
                lds[0] = issue_l(0)
                for q in range(8):
                    if q + 1 < 8:
                        if q >= 1:
                            sca[q - 1].wait()
                        lds[q + 1] = issue_l(q + 1)
                    lds[q].wait()
                    sca[q] = pltpu.async_copy(
                        buf.at[q % 2], acc.at[idx.at[q]], sems, add=True)
                sca[6].wait()
                sca[7].wait()
                return carry
            return body

        @pl.when(c == 0)
        def _():
            lax.fori_loop(0, ew2 // 1024, mk_body(mv), 0)

        @pl.when(c == 1)
        def _():
            lax.fori_loop(0, ew2 // 1024, mk_body(gv), 0)

        plsc.subcore_barrier()
        pltpu.sync_copy(acc.at[pl.ds(srw, rw)],
                        out.at[pl.ds(pl.multiple_of(c * npad + srw, 8), rw)])

    return sk


def _node_body(x_ref, pp_ref, pm_ref, pg_ref, wn1a_ref, wn1b_ref, bn1_ref,
               wn2_ref, bn2_ref, xo_ref, po_ref):
    pg = pg_ref[...]
    lane = lax.broadcasted_iota(jnp.int32, pg.shape, 1)
    deg = jnp.sum(jnp.where(lane == 3, pg, 0.0), axis=1, keepdims=True)
    deg = jnp.maximum(deg, 1.0)
    msum = pm_ref[...] / deg
    hn = _silu(jnp.dot(x_ref[...], wn1a_ref[...],
                       preferred_element_type=jnp.float32)
               + jnp.dot(msum, wn1b_ref[...],
                         preferred_element_type=jnp.float32)
               + bn1_ref[...])
    xo_ref[...] = (jnp.dot(hn, wn2_ref[...], preferred_element_type=jnp.float32)
                   + bn2_ref[...])
    pg16 = pg[:, :16]
    lane16 = lax.broadcasted_iota(jnp.int32, pg16.shape, 1)
    po_ref[...] = pp_ref[...] + jnp.where(lane16 < 3, pg16 / deg, 0.0)


def _node_call(x, pp, pm, pg, wn1a, wn1b, bn1r, wn2, bn2r, bn):
    n, d = x.shape
    h = wn2.shape[0]
    return pl.pallas_call(
        _node_body,
        grid=(n // bn,),
        in_specs=[
            pl.BlockSpec((bn, d), lambda i: (i, 0)),
            pl.BlockSpec((bn, 16), lambda i: (i, 0)),
            pl.BlockSpec((bn, d), lambda i: (i, 0)),
            pl.BlockSpec((bn, d), lambda i: (i, 0)),
            pl.BlockSpec((d, h), lambda i: (0, 0)),
            pl.BlockSpec((h, h), lambda i: (0, 0)),
            pl.BlockSpec((1, h), lambda i: (0, 0)),
            pl.BlockSpec((h, d), lambda i: (0, 0)),
            pl.BlockSpec((1, d), lambda i: (0, 0)),
        ],
        out_specs=[
            pl.BlockSpec((bn, d), lambda i: (i, 0)),
            pl.BlockSpec((bn, 16), lambda i: (i, 0)),
        ],
        out_shape=[
            jax.ShapeDtypeStruct((n, d), jnp.float32),
            jax.ShapeDtypeStruct((n, 16), jnp.float32),
        ],
    )(x, pp, pm, pg, wn1a, wn1b, bn1r, wn2, bn2r)


def kernel(x, pos, edge_index, edge_attr, W1, b1, W2, b2, Wn1, bn1, Wn2, bn2,
           Wc, bc):
    n, d = x.shape
    e = edge_index.shape[1]
    h = W2.shape[0]

    ew = -(-e // (NWK * 1024)) * 1024
    epad = ew * NWK
    npad = -(-n // 128) * 128
    if npad == n:
        npad += 128
    rw = npad // NS
    np4 = -(-(4 * n) // 128) * 128

    pp = jnp.pad(pos, ((0, 0), (0, 16 - pos.shape[1])))
    posf = jnp.pad(pos, ((0, 0), (0, 1))).reshape(-1)
    posf = jnp.pad(posf, (0, np4 - posf.shape[0]))
    src = edge_index[0]
    dst = edge_index[1]
    pe = epad - e
    dst_g = jnp.concatenate([dst, jnp.zeros((pe,), jnp.int32)]).reshape(-1, 128)
    src_g = jnp.concatenate([src, jnp.zeros((pe,), jnp.int32)]).reshape(-1, 128)
    dst_s = jnp.concatenate(
        [dst, jnp.full((pe,), npad - 1, jnp.int32)]).reshape(-1, 128)
    eap = jnp.pad(edge_attr, ((0, pe), (0, 0)))
    w1a = W1[:d]
    w1b = W1[d:2 * d]
    w1c = W1[2 * d:2 * d + 1]
    w1d = W1[2 * d + 1:]
    b1r = b1.reshape(1, h)
    b2r = b2.reshape(1, h)
    wcr = Wc.reshape(1, h)
    bcr = bc.reshape(1, 1)
    wn1a = Wn1[:d]
    wn1b = Wn1[d:]
    bn1r = bn1.reshape(1, h)
    bn2r = bn2.reshape(1, d)
    zer = jnp.zeros((npad, d), jnp.float32)

    bn = _pick_block(n, (1024, 1000, 512, 500, 256, 250, 200, 128, 8))
    be = _pick_block(epad, (2048, 1024, 512, 256, 128))

    ta, tb = _prep_call(x, w1a, w1b, bn)
    chp = epad // NS // 1024
    ga, gb, geo = _make_gather(epad, d, np4, chp * 13 // 20)(
        ta, tb, dst_g, src_g, posf)
    m, gd = _edge_call(ga, gb, geo, eap, w1c, w1d, b1r, W2, b2r, wcr, bcr, be)
    parts = _make_scatter(epad, npad, rw, d)(m, gd, dst_s, zer)
    pm = parts[:n]
    pg = parts[npad:npad + n]
    xo, po = _node_call(x, pp, pm, pg, wn1a, wn1b, bn1r, Wn2, bn2r, bn)
    return (xo, po[:, :3])


# speedup vs baseline: 1.3031x; 1.0379x over previous
import functools

import jax
import jax.numpy as jnp
from jax import lax
from jax.experimental import pallas as pl
from jax.experimental.pallas import tpu as pltpu
from jax.experimental.pallas import tpu_sc as plsc

NC = 2
NS = 16
NWK = NC * NS


def _silu(v):
    return v * jax.nn.sigmoid(v)


def _pick_block(n, cands):
    for c in cands:
        if n % c == 0:
            return c
    return n


def _prep_body(x_ref, w1a_ref, w1b_ref, ta_ref, tb_ref):
    xb = x_ref[...]
    ta_ref[...] = jnp.dot(xb, w1a_ref[...], preferred_element_type=jnp.float32)
    tb_ref[...] = jnp.dot(xb, w1b_ref[...], preferred_element_type=jnp.float32)


def _prep_call(x, w1a, w1b, bn):
    n, d = x.shape
    return pl.pallas_call(
        _prep_body,
        grid=(n // bn,),
        in_specs=[
            pl.BlockSpec((bn, d), lambda i: (i, 0)),
            pl.BlockSpec((d, d), lambda i: (0, 0)),
            pl.BlockSpec((d, d), lambda i: (0, 0)),
        ],
        out_specs=[
            pl.BlockSpec((bn, d), lambda i: (i, 0)),
            pl.BlockSpec((bn, d), lambda i: (i, 0)),
        ],
        out_shape=[
            jax.ShapeDtypeStruct((n, d), jnp.float32),
            jax.ShapeDtypeStruct((n, d), jnp.float32),
        ],
    )(x, w1a, w1b)


def _make_gather(epad, d, np4, ch0):
    chp = epad // NS // 1024
    ch1 = chp - ch0
    mesh = plsc.VectorSubcoreMesh(
        core_axis_name="c", subcore_axis_name="s", num_cores=NC, num_subcores=NS)

    @functools.partial(
        pl.kernel,
        out_type=(jax.ShapeDtypeStruct((epad, d), jnp.float32),
                  jax.ShapeDtypeStruct((epad, d), jnp.float32),
                  jax.ShapeDtypeStruct((4, epad), jnp.float32)),
        mesh=mesh,
        scratch_types=[
            pltpu.VMEM((8, 128), jnp.int32),
            pltpu.VMEM((8, 128), jnp.int32),
            pltpu.VMEM((2, 128, 128), jnp.float32),
            pltpu.VMEM((2, 128, 128), jnp.float32),
            pltpu.VMEM((np4,), jnp.float32),
            pltpu.VMEM((4, 1024), jnp.float32),
            pltpu.SemaphoreType.DMA,
            pltpu.SemaphoreType.DMA,
            pltpu.SemaphoreType.DMA,
        ],
        compiler_params=pltpu.CompilerParams(needs_layout_passes=False),
    )
    def gk(ta, tb, dsti, srci, posf, ga, gb, geo,
           idxd, idxs, bufa, bufb, posv, gbuf, sema, semb, semw):
        c = lax.axis_index("c")
        s = lax.axis_index("s")
        base = s * (chp * 1024) + c * (ch0 * 1024)
        nch = jnp.where(c == 0, ch0, ch1)
        pltpu.sync_copy(posf, posv)

        def body(i, carry):
            off = pl.multiple_of(base + i * 1024, 1024)
            r0 = pl.multiple_of(off // 128, 8)
            pltpu.sync_copy(dsti.at[pl.ds(r0, 8)], idxd)
            pltpu.sync_copy(srci.at[pl.ds(r0, 8)], idxs)
            def issue_g(q):
                b = q % 2
                return (
                    pltpu.async_copy(ta.at[idxd.at[q]], bufa.at[b], sema),
                    pltpu.async_copy(tb.at[idxs.at[q]], bufb.at[b], semb),
                )

            def issue_w(q):
                b = q % 2
                return (
                    pltpu.async_copy(
                        bufa.at[b], ga.at[pl.ds(off + q * 128, 128)], semw),
                    pltpu.async_copy(
                        bufb.at[b], gb.at[pl.ds(off + q * 128, 128)], semw),
                )

            gth = [None] * 8
            wbk = [None] * 8
            gth[0] = issue_g(0)
            for j in range(8):
                for kk in range(8):
                    lq = kk * 16
                    p = j * 128 + lq
                    di = idxd[j, pl.ds(lq, 16)] * 4
                    si = idxs[j, pl.ds(lq, 16)] * 4
                    dx = (plsc.load_gather(posv, [di])
                          - plsc.load_gather(posv, [si]))
                    dy = (plsc.load_gather(posv, [di + 1])
                          - plsc.load_gather(posv, [si + 1]))
                    dz = (plsc.load_gather(posv, [di + 2])
                          - plsc.load_gather(posv, [si + 2]))
                    gbuf[0, pl.ds(p, 16)] = dx
                    gbuf[1, pl.ds(p, 16)] = dy
                    gbuf[2, pl.ds(p, 16)] = dz
                    gbuf[3, pl.ds(p, 16)] = dx * dx + dy * dy + dz * dz
            pltpu.sync_copy(gbuf, geo.at[:, pl.ds(off, 1024)])
            for q in range(8):
                if q + 1 < 8:
                    if q >= 1:
                        wbk[q - 1][0].wait()
                        wbk[q - 1][1].wait()
                    gth[q + 1] = issue_g(q + 1)
                gth[q][0].wait()
                gth[q][1].wait()
                wbk[q] = issue_w(q)
            wbk[6][0].wait()
            wbk[6][1].wait()
            wbk[7][0].wait()
            wbk[7][1].wait()
            return carry

        lax.fori_loop(0, nch, body, 0)

    return gk


def _edge_body(ga_ref, gb_ref, geo_ref, ea_ref, w1c_ref, w1d_ref, b1_ref,
               w2_ref, b2_ref, wc_ref, bc_ref, m_ref, gd_ref):
    geo = geo_ref[...]
    i4a = lax.broadcasted_iota(jnp.int32, (4, 4), 0)
    i4b = lax.broadcasted_iota(jnp.int32, (4, 4), 1)
    eye4 = (i4a == i4b).astype(jnp.float32)
    d4 = lax.dot_general(geo, eye4, (((0,), (0,)), ((), ())),
                         preferred_element_type=jnp.float32)
    r2 = d4[:, 3:4]
    rinv = lax.rsqrt(r2 + 1e-8)
    pre = (ga_ref[...] + gb_ref[...] + r2 * w1c_ref[...] + b1_ref[...]
           + jnp.dot(ea_ref[...], w1d_ref[...],
                     preferred_element_type=jnp.float32))
    h = _silu(pre)
    m = _silu(jnp.dot(h, w2_ref[...], preferred_element_type=jnp.float32)
              + b2_ref[...])
    m_ref[...] = m
    gamma = jnp.sum(m * wc_ref[...], axis=1, keepdims=True) + bc_ref[...]
    lane4 = lax.broadcasted_iota(jnp.int32, d4.shape, 1)
    gd4 = jnp.where(lane4 == 3, 1.0, gamma * rinv * d4)
    be = d4.shape[0]
    gd_ref[...] = jnp.concatenate(
        [gd4, jnp.zeros((be, 124), jnp.float32)], axis=1)


def _edge_call(ga, gb, geo, eap, w1c, w1d, b1r, w2, b2r, wcr, bcr, be):
    epad, d = ga.shape
    ed = eap.shape[1]
    h = w2.shape[0]
    return pl.pallas_call(
        _edge_body,
        grid=(epad // be,),
        in_specs=[
            pl.BlockSpec((be, d), lambda i: (i, 0)),
            pl.BlockSpec((be, d), lambda i: (i, 0)),
            pl.BlockSpec((4, be), lambda i: (0, i)),
            pl.BlockSpec((be, ed), lambda i: (i, 0)),
            pl.BlockSpec((1, h), lambda i: (0, 0)),
            pl.BlockSpec((ed, h), lambda i: (0, 0)),
            pl.BlockSpec((1, h), lambda i: (0, 0)),
            pl.BlockSpec((h, h), lambda i: (0, 0)),
            pl.BlockSpec((1, h), lambda i: (0, 0)),
            pl.BlockSpec((1, h), lambda i: (0, 0)),
            pl.BlockSpec((1, 1), lambda i: (0, 0)),
        ],
        out_specs=[
            pl.BlockSpec((be, d), lambda i: (i, 0)),
            pl.BlockSpec((be, d), lambda i: (i, 0)),
        ],
        out_shape=[
            jax.ShapeDtypeStruct((epad, d), jnp.float32),
            jax.ShapeDtypeStruct((epad, d), jnp.float32),
        ],
    )(ga, gb, geo, eap, w1c, w1d, b1r, w2, b2r, wcr, bcr)


def _make_scatter(epad, npad, rw, d):
    ew2 = epad // NS
    mesh = plsc.VectorSubcoreMesh(
        core_axis_name="c", subcore_axis_name="s", num_cores=NC, num_subcores=NS)

    @functools.partial(
        pl.kernel,
        out_type=jax.ShapeDtypeStruct((NC * npad, d), jnp.float32),
        mesh=mesh,
        scratch_types=[
            pltpu.VMEM((8, 128), jnp.int32),
            pltpu.VMEM((2, 128, d), jnp.float32),
            pltpu.VMEM_SHARED((npad, d), jnp.float32),
            pltpu.SemaphoreType.DMA,
            pltpu.SemaphoreType.DMA,
        ],
    )
    def sk(mv, gv, dsti, zer, out, idx, buf, acc, seml, sems):
        c = lax.axis_index("c")
        s = lax.axis_index("s")
        base = s * ew2
        srw = pl.multiple_of(s * rw, 8)
        pltpu.sync_copy(zer.at[pl.ds(srw, rw)], acc.at[pl.ds(srw, rw)])
        plsc.subcore_barrier()

        def mk_body(data):
            def body(i, carry):
                off = pl.multiple_of(base + i * 1024, 1024)
                r0 = pl.multiple_of(off // 128, 8)
                pltpu.sync_copy(dsti.at[pl.ds(r0, 8)], idx)

                def issue_l(q):
                    return pltpu.async_copy(
                        data.at[pl.ds(off + q * 128, 128)],
                        buf.at[q % 2], seml)

                lds = [None] * 8
                sca = [None] * 8
                lds[0] = issue_l(0)
                for q in range(8):
                    if q + 1 < 8:
                        if q >= 1:
                            sca[q - 1].wait()
                        lds[q + 1] = issue_l(q + 1)
                    lds[q].wait()
                    sca[q] = pltpu.async_copy(
                        buf.at[q % 2], acc.at[idx.at[q]], sems, add=True)
                sca[6].wait()
                sca[7].wait()
                return carry
            return body

        @pl.when(c == 0)
        def _():
            lax.fori_loop(0, ew2 // 1024, mk_body(mv), 0)

        @pl.when(c == 1)
        def _():
            lax.fori_loop(0, ew2 // 1024, mk_body(gv), 0)

        plsc.subcore_barrier()
        pltpu.sync_copy(acc.at[pl.ds(srw, rw)],
                        out.at[pl.ds(pl.multiple_of(c * npad + srw, 8), rw)])

    return sk


def _node_body(x_ref, pp_ref, pm_ref, pg_ref, wn1a_ref, wn1b_ref, bn1_ref,
               wn2_ref, bn2_ref, xo_ref, po_ref):
    pg = pg_ref[...]
    lane = lax.broadcasted_iota(jnp.int32, pg.shape, 1)
    deg = jnp.sum(jnp.where(lane == 3, pg, 0.0), axis=1, keepdims=True)
    deg = jnp.maximum(deg, 1.0)
    msum = pm_ref[...] / deg
    hn = _silu(jnp.dot(x_ref[...], wn1a_ref[...],
                       preferred_element_type=jnp.float32)
               + jnp.dot(msum, wn1b_ref[...],
                         preferred_element_type=jnp.float32)
               + bn1_ref[...])
    xo_ref[...] = (jnp.dot(hn, wn2_ref[...], preferred_element_type=jnp.float32)
                   + bn2_ref[...])
    pg16 = pg[:, :16]
    lane16 = lax.broadcasted_iota(jnp.int32, pg16.shape, 1)
    po_ref[...] = pp_ref[...] + jnp.where(lane16 < 3, pg16 / deg, 0.0)


def _node_call(x, pp, pm, pg, wn1a, wn1b, bn1r, wn2, bn2r, bn):
    n, d = x.shape
    h = wn2.shape[0]
    return pl.pallas_call(
        _node_body,
        grid=(n // bn,),
        in_specs=[
            pl.BlockSpec((bn, d), lambda i: (i, 0)),
            pl.BlockSpec((bn, 16), lambda i: (i, 0)),
            pl.BlockSpec((bn, d), lambda i: (i, 0)),
            pl.BlockSpec((bn, d), lambda i: (i, 0)),
            pl.BlockSpec((d, h), lambda i: (0, 0)),
            pl.BlockSpec((h, h), lambda i: (0, 0)),
            pl.BlockSpec((1, h), lambda i: (0, 0)),
            pl.BlockSpec((h, d), lambda i: (0, 0)),
            pl.BlockSpec((1, d), lambda i: (0, 0)),
        ],
        out_specs=[
            pl.BlockSpec((bn, d), lambda i: (i, 0)),
            pl.BlockSpec((bn, 16), lambda i: (i, 0)),
        ],
        out_shape=[
            jax.ShapeDtypeStruct((n, d), jnp.float32),
            jax.ShapeDtypeStruct((n, 16), jnp.float32),
        ],
    )(x, pp, pm, pg, wn1a, wn1b, bn1r, wn2, bn2r)


def kernel(x, pos, edge_index, edge_attr, W1, b1, W2, b2, Wn1, bn1, Wn2, bn2,
           Wc, bc):
    n, d = x.shape
    e = edge_index.shape[1]
    h = W2.shape[0]

    ew = -(-e // (NWK * 1024)) * 1024
    epad = ew * NWK
    npad = -(-n // 128) * 128
    if npad == n:
        npad += 128
    rw = npad // NS
    np4 = -(-(4 * n) // 128) * 128

    pp = jnp.pad(pos, ((0, 0), (0, 16 - pos.shape[1])))
    posf = jnp.pad(pos, ((0, 0), (0, 1))).reshape(-1)
    posf = jnp.pad(posf, (0, np4 - posf.shape[0]))
    src = edge_index[0]
    dst = edge_index[1]
    pe = epad - e
    dst_g = jnp.concatenate([dst, jnp.zeros((pe,), jnp.int32)]).reshape(-1, 128)
    src_g = jnp.concatenate([src, jnp.zeros((pe,), jnp.int32)]).reshape(-1, 128)
    dst_s = jnp.concatenate(
        [dst, jnp.full((pe,), npad - 1, jnp.int32)]).reshape(-1, 128)
    eap = jnp.pad(edge_attr, ((0, pe), (0, 0)))
    w1a = W1[:d]
    w1b = W1[d:2 * d]
    w1c = W1[2 * d:2 * d + 1]
    w1d = W1[2 * d + 1:]
    b1r = b1.reshape(1, h)
    b2r = b2.reshape(1, h)
    wcr = Wc.reshape(1, h)
    bcr = bc.reshape(1, 1)
    wn1a = Wn1[:d]
    wn1b = Wn1[d:]
    bn1r = bn1.reshape(1, h)
    bn2r = bn2.reshape(1, d)
    zer = jnp.zeros((npad, d), jnp.float32)

    bn = _pick_block(n, (1024, 1000, 512, 500, 256, 250, 200, 128, 8))
    be = _pick_block(epad, (4096, 2048, 1024, 512, 256, 128))

    ta, tb = _prep_call(x, w1a, w1b, bn)
    chp = epad // NS // 1024
    ga, gb, geo = _make_gather(epad, d, np4, chp * 13 // 20)(
        ta, tb, dst_g, src_g, posf)
    m, gd = _edge_call(ga, gb, geo, eap, w1c, w1d, b1r, W2, b2r, wcr, bcr, be)
    parts = _make_scatter(epad, npad, rw, d)(m, gd, dst_s, zer)
    pm = parts[:n]
    pg = parts[npad:npad + n]
    xo, po = _node_call(x, pp, pm, pg, wn1a, wn1b, bn1r, Wn2, bn2r, bn)
    return (xo, po[:, :3])


# speedup vs baseline: 1.3204x; 1.0133x over previous
import functools

import jax
import jax.numpy as jnp
from jax import lax
from jax.experimental import pallas as pl
from jax.experimental.pallas import tpu as pltpu
from jax.experimental.pallas import tpu_sc as plsc

NC = 2
NS = 16
NWK = NC * NS


def _silu(v):
    return v * jax.nn.sigmoid(v)


def _pick_block(n, cands):
    for c in cands:
        if n % c == 0:
            return c
    return n


def _prep_body(x_ref, w1a_ref, w1b_ref, ta_ref, tb_ref):
    xb = x_ref[...]
    ta_ref[...] = jnp.dot(xb, w1a_ref[...], preferred_element_type=jnp.float32)
    tb_ref[...] = jnp.dot(xb, w1b_ref[...], preferred_element_type=jnp.float32)


def _prep_call(x, w1a, w1b, bn):
    n, d = x.shape
    return pl.pallas_call(
        _prep_body,
        grid=(n // bn,),
        in_specs=[
            pl.BlockSpec((bn, d), lambda i: (i, 0)),
            pl.BlockSpec((d, d), lambda i: (0, 0)),
            pl.BlockSpec((d, d), lambda i: (0, 0)),
        ],
        out_specs=[
            pl.BlockSpec((bn, d), lambda i: (i, 0)),
            pl.BlockSpec((bn, d), lambda i: (i, 0)),
        ],
        out_shape=[
            jax.ShapeDtypeStruct((n, d), jnp.float32),
            jax.ShapeDtypeStruct((n, d), jnp.float32),
        ],
    )(x, w1a, w1b)


def _make_gather(epad, d, np4, ch0):
    chp = epad // NS // 1024
    ch1 = chp - ch0
    mesh = plsc.VectorSubcoreMesh(
        core_axis_name="c", subcore_axis_name="s", num_cores=NC, num_subcores=NS)

    @functools.partial(
        pl.kernel,
        out_type=(jax.ShapeDtypeStruct((epad, d), jnp.float32),
                  jax.ShapeDtypeStruct((epad, d), jnp.float32),
                  jax.ShapeDtypeStruct((4, epad), jnp.float32)),
        mesh=mesh,
        scratch_types=[
            pltpu.VMEM((8, 128), jnp.int32),
            pltpu.VMEM((8, 128), jnp.int32),
            pltpu.VMEM((2, 128, 128), jnp.float32),
            pltpu.VMEM((2, 128, 128), jnp.float32),
            pltpu.VMEM((np4,), jnp.float32),
            pltpu.VMEM((4, 1024), jnp.float32),
            pltpu.SemaphoreType.DMA,
            pltpu.SemaphoreType.DMA,
            pltpu.SemaphoreType.DMA,
        ],
        compiler_params=pltpu.CompilerParams(needs_layout_passes=False),
    )
    def gk(ta, tb, dsti, srci, posf, ga, gb, geo,
           idxd, idxs, bufa, bufb, posv, gbuf, sema, semb, semw):
        c = lax.axis_index("c")
        s = lax.axis_index("s")
        base = s * (chp * 1024) + c * (ch0 * 1024)
        nch = jnp.where(c == 0, ch0, ch1)
        pltpu.sync_copy(posf, posv)

        def body(i, carry):
            off = pl.multiple_of(base + i * 1024, 1024)
            r0 = pl.multiple_of(off // 128, 8)
            pltpu.sync_copy(dsti.at[pl.ds(r0, 8)], idxd)
            pltpu.sync_copy(srci.at[pl.ds(r0, 8)], idxs)
            def issue_g(q):
                b = q % 2
                return (
                    pltpu.async_copy(ta.at[idxd.at[q]], bufa.at[b], sema),
                    pltpu.async_copy(tb.at[idxs.at[q]], bufb.at[b], semb),
                )

            def issue_w(q):
                b = q % 2
                return (
                    pltpu.async_copy(
                        bufa.at[b], ga.at[pl.ds(off + q * 128, 128)], semw),
                    pltpu.async_copy(
                        bufb.at[b], gb.at[pl.ds(off + q * 128, 128)], semw),
                )

            gth = [None] * 8
            wbk = [None] * 8
            gth[0] = issue_g(0)
            for j in range(8):
                for kk in range(8):
                    lq = kk * 16
                    p = j * 128 + lq
                    di = idxd[j, pl.ds(lq, 16)] * 4
                    si = idxs[j, pl.ds(lq, 16)] * 4
                    dx = (plsc.load_gather(posv, [di])
                          - plsc.load_gather(posv, [si]))
                    dy = (plsc.load_gather(posv, [di + 1])
                          - plsc.load_gather(posv, [si + 1]))
                    dz = (plsc.load_gather(posv, [di + 2])
                          - plsc.load_gather(posv, [si + 2]))
                    gbuf[0, pl.ds(p, 16)] = dx
                    gbuf[1, pl.ds(p, 16)] = dy
                    gbuf[2, pl.ds(p, 16)] = dz
                    gbuf[3, pl.ds(p, 16)] = dx * dx + dy * dy + dz * dz
            pltpu.sync_copy(gbuf, geo.at[:, pl.ds(off, 1024)])
            for q in range(8):
                if q + 1 < 8:
                    if q >= 1:
                        wbk[q - 1][0].wait()
                        wbk[q - 1][1].wait()
                    gth[q + 1] = issue_g(q + 1)
                gth[q][0].wait()
                gth[q][1].wait()
                wbk[q] = issue_w(q)
            wbk[6][0].wait()
            wbk[6][1].wait()
            wbk[7][0].wait()
            wbk[7][1].wait()
            return carry

        lax.fori_loop(0, nch, body, 0)

    return gk


def _edge_body(ga_ref, gb_ref, geo_ref, ea_ref, w1c_ref, w1d_ref, b1_ref,
               w2_ref, b2_ref, wc_ref, bc_ref, m_ref, gd_ref):
    geo = geo_ref[...]
    i4a = lax.broadcasted_iota(jnp.int32, (4, 4), 0)
    i4b = lax.broadcasted_iota(jnp.int32, (4, 4), 1)
    eye4 = (i4a == i4b).astype(jnp.float32)
    d4 = lax.dot_general(geo, eye4, (((0,), (0,)), ((), ())),
                         preferred_element_type=jnp.float32)
    r2 = d4[:, 3:4]
    rinv = lax.rsqrt(r2 + 1e-8)
    pre = (ga_ref[...] + gb_ref[...] + r2 * w1c_ref[...] + b1_ref[...]
           + jnp.dot(ea_ref[...], w1d_ref[...],
                     preferred_element_type=jnp.float32))
    h = _silu(pre)
    m = _silu(jnp.dot(h, w2_ref[...], preferred_element_type=jnp.float32)
              + b2_ref[...])
    m_ref[...] = m
    gamma = jnp.sum(m * wc_ref[...], axis=1, keepdims=True) + bc_ref[...]
    lane4 = lax.broadcasted_iota(jnp.int32, d4.shape, 1)
    gd4 = jnp.where(lane4 == 3, 1.0, gamma * rinv * d4)
    be = d4.shape[0]
    gd_ref[...] = jnp.concatenate(
        [gd4, jnp.zeros((be, 124), jnp.float32)], axis=1)


def _edge_call(ga, gb, geo, eap, w1c, w1d, b1r, w2, b2r, wcr, bcr, be):
    epad, d = ga.shape
    ed = eap.shape[1]
    h = w2.shape[0]
    return pl.pallas_call(
        _edge_body,
        grid=(epad // be,),
        in_specs=[
            pl.BlockSpec((be, d), lambda i: (i, 0)),
            pl.BlockSpec((be, d), lambda i: (i, 0)),
            pl.BlockSpec((4, be), lambda i: (0, i)),
            pl.BlockSpec((be, ed), lambda i: (i, 0)),
            pl.BlockSpec((1, h), lambda i: (0, 0)),
            pl.BlockSpec((ed, h), lambda i: (0, 0)),
            pl.BlockSpec((1, h), lambda i: (0, 0)),
            pl.BlockSpec((h, h), lambda i: (0, 0)),
            pl.BlockSpec((1, h), lambda i: (0, 0)),
            pl.BlockSpec((1, h), lambda i: (0, 0)),
            pl.BlockSpec((1, 1), lambda i: (0, 0)),
        ],
        out_specs=[
            pl.BlockSpec((be, d), lambda i: (i, 0)),
            pl.BlockSpec((be, d), lambda i: (i, 0)),
        ],
        out_shape=[
            jax.ShapeDtypeStruct((epad, d), jnp.float32),
            jax.ShapeDtypeStruct((epad, d), jnp.float32),
        ],
    )(ga, gb, geo, eap, w1c, w1d, b1r, w2, b2r, wcr, bcr)


def _make_scatter(epad, npad, rw, d):
    ew2 = epad // NS
    mesh = plsc.VectorSubcoreMesh(
        core_axis_name="c", subcore_axis_name="s", num_cores=NC, num_subcores=NS)

    @functools.partial(
        pl.kernel,
        out_type=jax.ShapeDtypeStruct((NC * npad, d), jnp.float32),
        mesh=mesh,
        scratch_types=[
            pltpu.VMEM((8, 128), jnp.int32),
            pltpu.VMEM((2, 128, d), jnp.float32),
            pltpu.VMEM_SHARED((npad, d), jnp.float32),
            pltpu.SemaphoreType.DMA,
            pltpu.SemaphoreType.DMA,
        ],
    )
    def sk(mv, gv, dsti, zer, out, idx, buf, acc, seml, sems):
        c = lax.axis_index("c")
        s = lax.axis_index("s")
        base = s * ew2
        srw = pl.multiple_of(s * rw, 8)
        pltpu.sync_copy(zer.at[pl.ds(srw, rw)], acc.at[pl.ds(srw, rw)])
        plsc.subcore_barrier()

        def mk_body(data):
            def body(i, carry):
                off = pl.multiple_of(base + i * 1024, 1024)
                r0 = pl.multiple_of(off // 128, 8)
                pltpu.sync_copy(dsti.at[pl.ds(r0, 8)], idx)

                def issue_l(q):
                    return pltpu.async_copy(
                        data.at[pl.ds(off + q * 128, 128)],
                        buf.at[q % 2], seml)

                lds = [None] * 8
                sca = [None] * 8
                lds[0] = issue_l(0)
                for q in range(8):
                    if q + 1 < 8:
                        if q >= 1:
                            sca[q - 1].wait()
                        lds[q + 1] = issue_l(q + 1)
                    lds[q].wait()
                    sca[q] = pltpu.async_copy(
                        buf.at[q % 2], acc.at[idx.at[q]], sems, add=True)
                sca[6].wait()
                sca[7].wait()
                return carry
            return body

        @pl.when(c == 0)
        def _():
            lax.fori_loop(0, ew2 // 1024, mk_body(mv), 0)

        @pl.when(c == 1)
        def _():
            lax.fori_loop(0, ew2 // 1024, mk_body(gv), 0)

        plsc.subcore_barrier()
        pltpu.sync_copy(acc.at[pl.ds(srw, rw)],
                        out.at[pl.ds(pl.multiple_of(c * npad + srw, 8), rw)])

    return sk


def _node_body(x_ref, pp_ref, pm_ref, pg_ref, wn1a_ref, wn1b_ref, bn1_ref,
               wn2_ref, bn2_ref, xo_ref, po_ref):
    pg = pg_ref[...]
    lane = lax.broadcasted_iota(jnp.int32, pg.shape, 1)
    deg = jnp.sum(jnp.where(lane == 3, pg, 0.0), axis=1, keepdims=True)
    deg = jnp.maximum(deg, 1.0)
    msum = pm_ref[...] / deg
    hn = _silu(jnp.dot(x_ref[...], wn1a_ref[...],
                       preferred_element_type=jnp.float32)
               + jnp.dot(msum, wn1b_ref[...],
                         preferred_element_type=jnp.float32)
               + bn1_ref[...])
    xo_ref[...] = (jnp.dot(hn, wn2_ref[...], preferred_element_type=jnp.float32)
                   + bn2_ref[...])
    pg16 = pg[:, :16]
    lane16 = lax.broadcasted_iota(jnp.int32, pg16.shape, 1)
    po_ref[...] = pp_ref[...] + jnp.where(lane16 < 3, pg16 / deg, 0.0)


def _node_call(x, pp, pm, pg, wn1a, wn1b, bn1r, wn2, bn2r, bn):
    n, d = x.shape
    h = wn2.shape[0]
    return pl.pallas_call(
        _node_body,
        grid=(n // bn,),
        in_specs=[
            pl.BlockSpec((bn, d), lambda i: (i, 0)),
            pl.BlockSpec((bn, 16), lambda i: (i, 0)),
            pl.BlockSpec((bn, d), lambda i: (i, 0)),
            pl.BlockSpec((bn, d), lambda i: (i, 0)),
            pl.BlockSpec((d, h), lambda i: (0, 0)),
            pl.BlockSpec((h, h), lambda i: (0, 0)),
            pl.BlockSpec((1, h), lambda i: (0, 0)),
            pl.BlockSpec((h, d), lambda i: (0, 0)),
            pl.BlockSpec((1, d), lambda i: (0, 0)),
        ],
        out_specs=[
            pl.BlockSpec((bn, d), lambda i: (i, 0)),
            pl.BlockSpec((bn, 16), lambda i: (i, 0)),
        ],
        out_shape=[
            jax.ShapeDtypeStruct((n, d), jnp.float32),
            jax.ShapeDtypeStruct((n, 16), jnp.float32),
        ],
    )(x, pp, pm, pg, wn1a, wn1b, bn1r, wn2, bn2r)


def kernel(x, pos, edge_index, edge_attr, W1, b1, W2, b2, Wn1, bn1, Wn2, bn2,
           Wc, bc):
    n, d = x.shape
    e = edge_index.shape[1]
    h = W2.shape[0]

    ew = -(-e // (NWK * 1024)) * 1024
    epad = ew * NWK
    npad = -(-n // 128) * 128
    if npad == n:
        npad += 128
    rw = npad // NS
    np4 = -(-(4 * n) // 128) * 128

    pp = jnp.pad(pos, ((0, 0), (0, 16 - pos.shape[1])))
    posf = jnp.pad(pos, ((0, 0), (0, 1))).reshape(-1)
    posf = jnp.pad(posf, (0, np4 - posf.shape[0]))
    src = edge_index[0]
    dst = edge_index[1]
    pe = epad - e
    dst_g = jnp.concatenate([dst, jnp.zeros((pe,), jnp.int32)]).reshape(-1, 128)
    src_g = jnp.concatenate([src, jnp.zeros((pe,), jnp.int32)]).reshape(-1, 128)
    dst_s = jnp.concatenate(
        [dst, jnp.full((pe,), npad - 1, jnp.int32)]).reshape(-1, 128)
    eap = jnp.pad(edge_attr, ((0, pe), (0, 0)))
    w1a = W1[:d]
    w1b = W1[d:2 * d]
    w1c = W1[2 * d:2 * d + 1]
    w1d = W1[2 * d + 1:]
    b1r = b1.reshape(1, h)
    b2r = b2.reshape(1, h)
    wcr = Wc.reshape(1, h)
    bcr = bc.reshape(1, 1)
    wn1a = Wn1[:d]
    wn1b = Wn1[d:]
    bn1r = bn1.reshape(1, h)
    bn2r = bn2.reshape(1, d)
    zer = jnp.zeros((npad, d), jnp.float32)

    bn = _pick_block(n, (1024, 1000, 512, 500, 256, 250, 200, 128, 8))
    be = _pick_block(epad, (8192, 4096, 2048, 1024, 512, 256, 128))

    ta, tb = _prep_call(x, w1a, w1b, bn)
    chp = epad // NS // 1024
    ga, gb, geo = _make_gather(epad, d, np4, chp * 13 // 20)(
        ta, tb, dst_g, src_g, posf)
    m, gd = _edge_call(ga, gb, geo, eap, w1c, w1d, b1r, W2, b2r, wcr, bcr, be)
    parts = _make_scatter(epad, npad, rw, d)(m, gd, dst_s, zer)
    pm = parts[:n]
    pg = parts[npad:npad + n]
    xo, po = _node_call(x, pp, pm, pg, wn1a, wn1b, bn1r, Wn2, bn2r, bn)
    return (xo, po[:, :3])
